# Initial kernel scaffold; baseline (speedup 1.0000x reference)
import jax
import jax.numpy as jnp
from jax.experimental import pallas as pl


def kernel(decoder_features, decoder_pos, encoder_features, encoder_pos, encoder_labels, Wq, bq, Wk, bk, Wv, bv, W1, b1, W2, b2, gamma, beta):
    raise NotImplementedError("write your pallas kernel here")


# speedup vs baseline: 16.7067x; 16.7067x over previous
import math
import jax
import jax.numpy as jnp
from jax.experimental import pallas as pl

ROWS = 256
DEC_N = 4096
K = 16
SCALE = 1.0 / 16.0


def _precompute_kv(df_ref, Wk_ref, bk_ref, Wv_ref, bv_ref, K_ref, V_ref):
    df = df_ref[...]
    K_ref[...] = jnp.dot(df, Wk_ref[...], preferred_element_type=jnp.float32) + bk_ref[...]
    V_ref[...] = jnp.dot(df, Wv_ref[...], preferred_element_type=jnp.float32) + bv_ref[...]


def _main(epos_ref, efeat_ref, dpos_ref, dn_ref, K_ref, V_ref, Wq_ref, bq_ref,
          W1a_ref, W1b_ref, b1_ref, W2_ref, b2_ref, gamma_ref, beta_ref,
          out_ref):
    ef = efeat_ref[...]
    ep = epos_ref[...].astype(jnp.float32)
    dpt = dpos_ref[...].astype(jnp.float32)
    delta = (ep[:, 0:1] * dpt[0:1, :] + ep[:, 1:2] * dpt[1:2, :]
             + ep[:, 2:3] * dpt[2:3, :] + dn_ref[...])

    def body(_, t):
        return jnp.min(jnp.where(delta > t, delta, jnp.inf), axis=1,
                       keepdims=True)
    t0 = jnp.full((ROWS, 1), -jnp.inf, dtype=jnp.float32)
    t = jax.lax.fori_loop(0, K, body, t0)

    q = jnp.dot(ef, Wq_ref[...], preferred_element_type=jnp.float32) + bq_ref[...]
    s = jax.lax.dot_general(q, K_ref[...], (((1,), (1,)), ((), ())),
                            preferred_element_type=jnp.float32) * SCALE
    s = jnp.where(delta <= t, s, -jnp.inf)
    m = jnp.max(s, axis=1, keepdims=True)
    p = jnp.exp(s - m)
    denom = jnp.sum(p, axis=1, keepdims=True)
    agg = jnp.dot(p, V_ref[...], preferred_element_type=jnp.float32) / denom

    h = (jnp.dot(agg, W1a_ref[...], preferred_element_type=jnp.float32)
         + jnp.dot(ef, W1b_ref[...], preferred_element_type=jnp.float32)
         + b1_ref[...])
    h = jnp.maximum(h, 0.0)
    up = jnp.dot(h, W2_ref[...], preferred_element_type=jnp.float32) + b2_ref[...]
    mu = jnp.mean(up, axis=1, keepdims=True)
    c = up - mu
    var = jnp.mean(c * c, axis=1, keepdims=True)
    out_ref[...] = c * jax.lax.rsqrt(var + 1e-5) * gamma_ref[...] + beta_ref[...]


def kernel(decoder_features, decoder_pos, encoder_features, encoder_pos,
           encoder_labels, Wq, bq, Wk, bk, Wv, bv, W1, b1, W2, b2, gamma, beta):
    enc_n = encoder_features.shape[0]
    out_dim = W2.shape[1]

    dn_row = jnp.sum(decoder_pos * decoder_pos, axis=1)[None, :]
    dpos_aug = jnp.concatenate(
        [decoder_pos.T.astype(jnp.bfloat16),
         jnp.zeros((1, DEC_N), jnp.bfloat16)], axis=0)
    epos_aug = jnp.concatenate(
        [(-2.0 * encoder_pos).astype(jnp.bfloat16),
         jnp.zeros((enc_n, 1), jnp.bfloat16)], axis=1)
    W1a, W1b = W1[:out_dim], W1[out_dim:]
    row = lambda v: v.reshape(1, -1)

    K_dec, V_dec = pl.pallas_call(
        _precompute_kv,
        out_shape=[jax.ShapeDtypeStruct((DEC_N, out_dim), jnp.float32)] * 2,
    )(decoder_features, Wk, row(bk), Wv, row(bv))

    grid = enc_n // ROWS
    out = pl.pallas_call(
        _main,
        grid=(grid,),
        in_specs=[
            pl.BlockSpec((ROWS, 4), lambda i: (i, 0)),
            pl.BlockSpec((ROWS, 128), lambda i: (i, 0)),
            pl.BlockSpec((4, DEC_N), lambda i: (0, 0)),
            pl.BlockSpec((1, DEC_N), lambda i: (0, 0)),
            pl.BlockSpec((DEC_N, out_dim), lambda i: (0, 0)),
            pl.BlockSpec((DEC_N, out_dim), lambda i: (0, 0)),
            pl.BlockSpec((128, 256), lambda i: (0, 0)),
            pl.BlockSpec((1, 256), lambda i: (0, 0)),
            pl.BlockSpec((256, 256), lambda i: (0, 0)),
            pl.BlockSpec((128, 256), lambda i: (0, 0)),
            pl.BlockSpec((1, 256), lambda i: (0, 0)),
            pl.BlockSpec((256, 256), lambda i: (0, 0)),
            pl.BlockSpec((1, 256), lambda i: (0, 0)),
            pl.BlockSpec((1, 256), lambda i: (0, 0)),
            pl.BlockSpec((1, 256), lambda i: (0, 0)),
        ],
        out_specs=pl.BlockSpec((ROWS, out_dim), lambda i: (i, 0)),
        out_shape=jax.ShapeDtypeStruct((enc_n, out_dim), jnp.float32),
    )(epos_aug, encoder_features, dpos_aug, dn_row, K_dec, V_dec, Wq, row(bq),
      W1a, W1b, row(b1), W2, row(b2), row(gamma), row(beta))

    return (out, encoder_pos, encoder_labels)


# speedup vs baseline: 20.7728x; 1.2434x over previous
import math
import jax
import jax.numpy as jnp
from jax.experimental import pallas as pl

ROWS = 256
DEC_N = 4096
K = 16
SCALE = 1.0 / 16.0


def _precompute_kv(df_ref, Wk_ref, bk_ref, Wv_ref, bv_ref, K_ref, V_ref):
    df = df_ref[...]
    K_ref[...] = jnp.dot(df, Wk_ref[...], preferred_element_type=jnp.float32) + bk_ref[...]
    V_ref[...] = jnp.dot(df, Wv_ref[...], preferred_element_type=jnp.float32) + bv_ref[...]


def _main(epos_ref, efeat_ref, dpos_ref, dn_ref, K_ref, V_ref, Wq_ref, bq_ref,
          W1a_ref, W1b_ref, b1_ref, W2_ref, b2_ref, gamma_ref, beta_ref,
          out_ref):
    ef = efeat_ref[...]
    ep = epos_ref[...].astype(jnp.float32)
    dpt = dpos_ref[...].astype(jnp.float32)
    delta = (ep[:, 0:1] * dpt[0:1, :] + ep[:, 1:2] * dpt[1:2, :]
             + ep[:, 2:3] * dpt[2:3, :] + dn_ref[...])

    t = jnp.full((ROWS, 1), -jnp.inf, dtype=jnp.float32)
    for _ in range(K):
        t = jnp.min(jnp.where(delta > t, delta, jnp.inf), axis=1,
                    keepdims=True)

    q = jnp.dot(ef, Wq_ref[...], preferred_element_type=jnp.float32) + bq_ref[...]
    s = jax.lax.dot_general(
        q.astype(jnp.bfloat16), K_ref[...].astype(jnp.bfloat16),
        (((1,), (1,)), ((), ())),
        preferred_element_type=jnp.float32) * SCALE
    s = jnp.where(delta <= t, s, -jnp.inf)
    m = jnp.max(s, axis=1, keepdims=True)
    p = jnp.exp(s - m)
    denom = jnp.sum(p, axis=1, keepdims=True)
    agg = jnp.dot(p.astype(jnp.bfloat16), V_ref[...].astype(jnp.bfloat16),
                  preferred_element_type=jnp.float32) / denom

    h = (jnp.dot(agg.astype(jnp.bfloat16), W1a_ref[...].astype(jnp.bfloat16),
                 preferred_element_type=jnp.float32)
         + jnp.dot(ef.astype(jnp.bfloat16), W1b_ref[...].astype(jnp.bfloat16),
                   preferred_element_type=jnp.float32)
         + b1_ref[...])
    h = jnp.maximum(h, 0.0)
    up = jnp.dot(h.astype(jnp.bfloat16), W2_ref[...].astype(jnp.bfloat16),
                 preferred_element_type=jnp.float32) + b2_ref[...]
    mu = jnp.mean(up, axis=1, keepdims=True)
    c = up - mu
    var = jnp.mean(c * c, axis=1, keepdims=True)
    out_ref[...] = c * jax.lax.rsqrt(var + 1e-5) * gamma_ref[...] + beta_ref[...]


def kernel(decoder_features, decoder_pos, encoder_features, encoder_pos,
           encoder_labels, Wq, bq, Wk, bk, Wv, bv, W1, b1, W2, b2, gamma, beta):
    enc_n = encoder_features.shape[0]
    out_dim = W2.shape[1]

    dn_row = jnp.sum(decoder_pos * decoder_pos, axis=1)[None, :]
    dpos_aug = jnp.concatenate(
        [decoder_pos.T.astype(jnp.bfloat16),
         jnp.zeros((1, DEC_N), jnp.bfloat16)], axis=0)
    epos_aug = jnp.concatenate(
        [(-2.0 * encoder_pos).astype(jnp.bfloat16),
         jnp.zeros((enc_n, 1), jnp.bfloat16)], axis=1)
    W1a, W1b = W1[:out_dim], W1[out_dim:]
    row = lambda v: v.reshape(1, -1)

    K_dec, V_dec = pl.pallas_call(
        _precompute_kv,
        out_shape=[jax.ShapeDtypeStruct((DEC_N, out_dim), jnp.float32)] * 2,
    )(decoder_features, Wk, row(bk), Wv, row(bv))

    grid = enc_n // ROWS
    out = pl.pallas_call(
        _main,
        grid=(grid,),
        in_specs=[
            pl.BlockSpec((ROWS, 4), lambda i: (i, 0)),
            pl.BlockSpec((ROWS, 128), lambda i: (i, 0)),
            pl.BlockSpec((4, DEC_N), lambda i: (0, 0)),
            pl.BlockSpec((1, DEC_N), lambda i: (0, 0)),
            pl.BlockSpec((DEC_N, out_dim), lambda i: (0, 0)),
            pl.BlockSpec((DEC_N, out_dim), lambda i: (0, 0)),
            pl.BlockSpec((128, 256), lambda i: (0, 0)),
            pl.BlockSpec((1, 256), lambda i: (0, 0)),
            pl.BlockSpec((256, 256), lambda i: (0, 0)),
            pl.BlockSpec((128, 256), lambda i: (0, 0)),
            pl.BlockSpec((1, 256), lambda i: (0, 0)),
            pl.BlockSpec((256, 256), lambda i: (0, 0)),
            pl.BlockSpec((1, 256), lambda i: (0, 0)),
            pl.BlockSpec((1, 256), lambda i: (0, 0)),
            pl.BlockSpec((1, 256), lambda i: (0, 0)),
        ],
        out_specs=pl.BlockSpec((ROWS, out_dim), lambda i: (i, 0)),
        out_shape=jax.ShapeDtypeStruct((enc_n, out_dim), jnp.float32),
    )(epos_aug, encoder_features, dpos_aug, dn_row, K_dec, V_dec, Wq, row(bq),
      W1a, W1b, row(b1), W2, row(b2), row(gamma), row(beta))

    return (out, encoder_pos, encoder_labels)


# speedup vs baseline: 23.5061x; 1.1316x over previous
import math
import jax
import jax.numpy as jnp
from jax.experimental import pallas as pl

ROWS = 256
DEC_N = 4096
K = 16
SCALE = 1.0 / 16.0


def _precompute_kv(df_ref, Wk_ref, bk_ref, Wv_ref, bv_ref, K_ref, V_ref):
    df = df_ref[...]
    K_ref[...] = jnp.dot(df, Wk_ref[...], preferred_element_type=jnp.float32) + bk_ref[...]
    V_ref[...] = jnp.dot(df, Wv_ref[...], preferred_element_type=jnp.float32) + bv_ref[...]


def _main(epos_ref, efeat_ref, dpos_ref, dn_ref, K_ref, V_ref, Wq_ref, bq_ref,
          W1a_ref, W1b_ref, b1_ref, W2_ref, b2_ref, gamma_ref, beta_ref,
          out_ref):
    ef = efeat_ref[...]
    ep = epos_ref[...].astype(jnp.float32)
    dpt = dpos_ref[...].astype(jnp.float32)
    delta = (ep[:, 0:1] * dpt[0:1, :] + ep[:, 1:2] * dpt[1:2, :]
             + ep[:, 2:3] * dpt[2:3, :] + dn_ref[...])

    NCAND = 5
    lanes = [jnp.full((ROWS, 128), jnp.inf, dtype=jnp.float32)] * NCAND
    for c in range(DEC_N // 128):
        x = delta[:, 128 * c:128 * (c + 1)]
        for j in range(NCAND):
            lo = jnp.minimum(lanes[j], x)
            x = jnp.maximum(lanes[j], x)
            lanes[j] = lo
    cand = jnp.concatenate(lanes, axis=1)
    t = jnp.full((ROWS, 1), -jnp.inf, dtype=jnp.float32)
    for _ in range(K):
        t = jnp.min(jnp.where(cand > t, cand, jnp.inf), axis=1,
                    keepdims=True)
    cnt_strict = jnp.sum((delta < t).astype(jnp.float32), axis=1)
    bad = jnp.any(cnt_strict > 15.5)

    def _exact_t(_):
        tf = jnp.full((ROWS, 1), -jnp.inf, dtype=jnp.float32)
        for _ in range(K):
            tf = jnp.min(jnp.where(delta > tf, delta, jnp.inf), axis=1,
                         keepdims=True)
        return tf

    t = jax.lax.cond(bad, _exact_t, lambda _: t, None)

    q = jnp.dot(ef, Wq_ref[...], preferred_element_type=jnp.float32) + bq_ref[...]
    s = jax.lax.dot_general(
        q.astype(jnp.bfloat16), K_ref[...].astype(jnp.bfloat16),
        (((1,), (1,)), ((), ())),
        preferred_element_type=jnp.float32) * SCALE
    s = jnp.where(delta <= t, s, -jnp.inf)
    m = jnp.max(s, axis=1, keepdims=True)
    p = jnp.exp(s - m)
    denom = jnp.sum(p, axis=1, keepdims=True)
    agg = jnp.dot(p.astype(jnp.bfloat16), V_ref[...].astype(jnp.bfloat16),
                  preferred_element_type=jnp.float32) / denom

    h = (jnp.dot(agg.astype(jnp.bfloat16), W1a_ref[...].astype(jnp.bfloat16),
                 preferred_element_type=jnp.float32)
         + jnp.dot(ef.astype(jnp.bfloat16), W1b_ref[...].astype(jnp.bfloat16),
                   preferred_element_type=jnp.float32)
         + b1_ref[...])
    h = jnp.maximum(h, 0.0)
    up = jnp.dot(h.astype(jnp.bfloat16), W2_ref[...].astype(jnp.bfloat16),
                 preferred_element_type=jnp.float32) + b2_ref[...]
    mu = jnp.mean(up, axis=1, keepdims=True)
    c = up - mu
    var = jnp.mean(c * c, axis=1, keepdims=True)
    out_ref[...] = c * jax.lax.rsqrt(var + 1e-5) * gamma_ref[...] + beta_ref[...]


def kernel(decoder_features, decoder_pos, encoder_features, encoder_pos,
           encoder_labels, Wq, bq, Wk, bk, Wv, bv, W1, b1, W2, b2, gamma, beta):
    enc_n = encoder_features.shape[0]
    out_dim = W2.shape[1]

    dn_row = jnp.sum(decoder_pos * decoder_pos, axis=1)[None, :]
    dpos_aug = jnp.concatenate(
        [decoder_pos.T.astype(jnp.bfloat16),
         jnp.zeros((1, DEC_N), jnp.bfloat16)], axis=0)
    epos_aug = jnp.concatenate(
        [(-2.0 * encoder_pos).astype(jnp.bfloat16),
         jnp.zeros((enc_n, 1), jnp.bfloat16)], axis=1)
    W1a, W1b = W1[:out_dim], W1[out_dim:]
    row = lambda v: v.reshape(1, -1)

    K_dec, V_dec = pl.pallas_call(
        _precompute_kv,
        out_shape=[jax.ShapeDtypeStruct((DEC_N, out_dim), jnp.float32)] * 2,
    )(decoder_features, Wk, row(bk), Wv, row(bv))

    grid = enc_n // ROWS
    out = pl.pallas_call(
        _main,
        grid=(grid,),
        in_specs=[
            pl.BlockSpec((ROWS, 4), lambda i: (i, 0)),
            pl.BlockSpec((ROWS, 128), lambda i: (i, 0)),
            pl.BlockSpec((4, DEC_N), lambda i: (0, 0)),
            pl.BlockSpec((1, DEC_N), lambda i: (0, 0)),
            pl.BlockSpec((DEC_N, out_dim), lambda i: (0, 0)),
            pl.BlockSpec((DEC_N, out_dim), lambda i: (0, 0)),
            pl.BlockSpec((128, 256), lambda i: (0, 0)),
            pl.BlockSpec((1, 256), lambda i: (0, 0)),
            pl.BlockSpec((256, 256), lambda i: (0, 0)),
            pl.BlockSpec((128, 256), lambda i: (0, 0)),
            pl.BlockSpec((1, 256), lambda i: (0, 0)),
            pl.BlockSpec((256, 256), lambda i: (0, 0)),
            pl.BlockSpec((1, 256), lambda i: (0, 0)),
            pl.BlockSpec((1, 256), lambda i: (0, 0)),
            pl.BlockSpec((1, 256), lambda i: (0, 0)),
        ],
        out_specs=pl.BlockSpec((ROWS, out_dim), lambda i: (i, 0)),
        out_shape=jax.ShapeDtypeStruct((enc_n, out_dim), jnp.float32),
    )(epos_aug, encoder_features, dpos_aug, dn_row, K_dec, V_dec, Wq, row(bq),
      W1a, W1b, row(b1), W2, row(b2), row(gamma), row(beta))

    return (out, encoder_pos, encoder_labels)


# speedup vs baseline: 23.5480x; 1.0018x over previous
import math
import jax
import jax.numpy as jnp
from jax.experimental import pallas as pl
from jax.experimental.pallas import tpu as pltpu

ROWS = 256
DEC_N = 4096
K = 16
SCALE = 1.0 / 16.0


def _precompute_kv(df_ref, Wk_ref, bk_ref, Wv_ref, bv_ref, K_ref, V_ref):
    df = df_ref[...]
    K_ref[...] = jnp.dot(df, Wk_ref[...], preferred_element_type=jnp.float32) + bk_ref[...]
    V_ref[...] = jnp.dot(df, Wv_ref[...], preferred_element_type=jnp.float32) + bv_ref[...]


def _main(epos_ref, efeat_ref, dpos_ref, dn_ref, K_ref, V_ref, Wq_ref, bq_ref,
          W1a_ref, W1b_ref, b1_ref, W2_ref, b2_ref, gamma_ref, beta_ref,
          out_ref, t_ref):
    ef = efeat_ref[...]
    ep = epos_ref[...].astype(jnp.float32)
    dpt = dpos_ref[...].astype(jnp.float32)
    delta = (ep[:, 0:1] * dpt[0:1, :] + ep[:, 1:2] * dpt[1:2, :]
             + ep[:, 2:3] * dpt[2:3, :] + dn_ref[...])

    NCAND = 5
    lanes = [jnp.full((ROWS, 128), jnp.inf, dtype=jnp.float32)] * NCAND
    for c in range(DEC_N // 128):
        x = delta[:, 128 * c:128 * (c + 1)]
        for j in range(NCAND):
            lo = jnp.minimum(lanes[j], x)
            x = jnp.maximum(lanes[j], x)
            lanes[j] = lo
    cand = jnp.concatenate(lanes, axis=1)
    t = jnp.full((ROWS, 1), -jnp.inf, dtype=jnp.float32)
    for _ in range(K):
        t = jnp.min(jnp.where(cand > t, cand, jnp.inf), axis=1,
                    keepdims=True)
    cnt_strict = jnp.sum((delta < t).astype(jnp.float32), axis=1)
    bad = jnp.any(cnt_strict > 15.5)
    t_ref[...] = t

    @pl.when(bad)
    def _fallback():
        tf = jnp.full((ROWS, 1), -jnp.inf, dtype=jnp.float32)
        for _ in range(K):
            tf = jnp.min(jnp.where(delta > tf, delta, jnp.inf), axis=1,
                         keepdims=True)
        t_ref[...] = tf

    t = t_ref[...]

    q = jnp.dot(ef, Wq_ref[...], preferred_element_type=jnp.float32) + bq_ref[...]
    s = jax.lax.dot_general(
        q.astype(jnp.bfloat16), K_ref[...].astype(jnp.bfloat16),
        (((1,), (1,)), ((), ())),
        preferred_element_type=jnp.float32) * SCALE
    s = jnp.where(delta <= t, s, -jnp.inf)
    m = jnp.max(s, axis=1, keepdims=True)
    p = jnp.exp(s - m)
    denom = jnp.sum(p, axis=1, keepdims=True)
    agg = jnp.dot(p.astype(jnp.bfloat16), V_ref[...].astype(jnp.bfloat16),
                  preferred_element_type=jnp.float32) / denom

    h = (jnp.dot(agg.astype(jnp.bfloat16), W1a_ref[...].astype(jnp.bfloat16),
                 preferred_element_type=jnp.float32)
         + jnp.dot(ef.astype(jnp.bfloat16), W1b_ref[...].astype(jnp.bfloat16),
                   preferred_element_type=jnp.float32)
         + b1_ref[...])
    h = jnp.maximum(h, 0.0)
    up = jnp.dot(h.astype(jnp.bfloat16), W2_ref[...].astype(jnp.bfloat16),
                 preferred_element_type=jnp.float32) + b2_ref[...]
    mu = jnp.mean(up, axis=1, keepdims=True)
    c = up - mu
    var = jnp.mean(c * c, axis=1, keepdims=True)
    out_ref[...] = c * jax.lax.rsqrt(var + 1e-5) * gamma_ref[...] + beta_ref[...]


def kernel(decoder_features, decoder_pos, encoder_features, encoder_pos,
           encoder_labels, Wq, bq, Wk, bk, Wv, bv, W1, b1, W2, b2, gamma, beta):
    enc_n = encoder_features.shape[0]
    out_dim = W2.shape[1]

    dn_row = jnp.sum(decoder_pos * decoder_pos, axis=1)[None, :]
    dpos_aug = jnp.concatenate(
        [decoder_pos.T.astype(jnp.bfloat16),
         jnp.zeros((1, DEC_N), jnp.bfloat16)], axis=0)
    epos_aug = jnp.concatenate(
        [(-2.0 * encoder_pos).astype(jnp.bfloat16),
         jnp.zeros((enc_n, 1), jnp.bfloat16)], axis=1)
    W1a, W1b = W1[:out_dim], W1[out_dim:]
    row = lambda v: v.reshape(1, -1)

    K_dec, V_dec = pl.pallas_call(
        _precompute_kv,
        out_shape=[jax.ShapeDtypeStruct((DEC_N, out_dim), jnp.float32)] * 2,
    )(decoder_features, Wk, row(bk), Wv, row(bv))

    grid = enc_n // ROWS
    out = pl.pallas_call(
        _main,
        grid=(grid,),
        in_specs=[
            pl.BlockSpec((ROWS, 4), lambda i: (i, 0)),
            pl.BlockSpec((ROWS, 128), lambda i: (i, 0)),
            pl.BlockSpec((4, DEC_N), lambda i: (0, 0)),
            pl.BlockSpec((1, DEC_N), lambda i: (0, 0)),
            pl.BlockSpec((DEC_N, out_dim), lambda i: (0, 0)),
            pl.BlockSpec((DEC_N, out_dim), lambda i: (0, 0)),
            pl.BlockSpec((128, 256), lambda i: (0, 0)),
            pl.BlockSpec((1, 256), lambda i: (0, 0)),
            pl.BlockSpec((256, 256), lambda i: (0, 0)),
            pl.BlockSpec((128, 256), lambda i: (0, 0)),
            pl.BlockSpec((1, 256), lambda i: (0, 0)),
            pl.BlockSpec((256, 256), lambda i: (0, 0)),
            pl.BlockSpec((1, 256), lambda i: (0, 0)),
            pl.BlockSpec((1, 256), lambda i: (0, 0)),
            pl.BlockSpec((1, 256), lambda i: (0, 0)),
        ],
        out_specs=pl.BlockSpec((ROWS, out_dim), lambda i: (i, 0)),
        out_shape=jax.ShapeDtypeStruct((enc_n, out_dim), jnp.float32),
        scratch_shapes=[pltpu.VMEM((ROWS, 1), jnp.float32)],
    )(epos_aug, encoder_features, dpos_aug, dn_row, K_dec, V_dec, Wq, row(bq),
      W1a, W1b, row(b1), W2, row(b2), row(gamma), row(beta))

    return (out, encoder_pos, encoder_labels)


# speedup vs baseline: 24.4235x; 1.0372x over previous
import math
import jax
import jax.numpy as jnp
from jax.experimental import pallas as pl
from jax.experimental.pallas import tpu as pltpu

ROWS = 512
DEC_N = 4096
K = 16
SCALE = 1.0 / 16.0


def _precompute_kv(df_ref, Wk_ref, bk_ref, Wv_ref, bv_ref, K_ref, V_ref):
    df = df_ref[...]
    K_ref[...] = jnp.dot(df, Wk_ref[...], preferred_element_type=jnp.float32) + bk_ref[...]
    V_ref[...] = jnp.dot(df, Wv_ref[...], preferred_element_type=jnp.float32) + bv_ref[...]


def _main(epos_ref, efeat_ref, dpos_ref, dn_ref, K_ref, V_ref, Wq_ref, bq_ref,
          W1a_ref, W1b_ref, b1_ref, W2_ref, b2_ref, gamma_ref, beta_ref,
          out_ref, t_ref):
    ef = efeat_ref[...]
    ep = epos_ref[...].astype(jnp.float32)
    dpt = dpos_ref[...].astype(jnp.float32)
    delta = (ep[:, 0:1] * dpt[0:1, :] + ep[:, 1:2] * dpt[1:2, :]
             + ep[:, 2:3] * dpt[2:3, :] + dn_ref[...])

    q = jnp.dot(ef, Wq_ref[...], preferred_element_type=jnp.float32) + bq_ref[...]
    s = jax.lax.dot_general(
        q.astype(jnp.bfloat16), K_ref[...].astype(jnp.bfloat16),
        (((1,), (1,)), ((), ())),
        preferred_element_type=jnp.float32) * SCALE

    NCAND = 5
    lanes = [jnp.full((ROWS, 128), jnp.inf, dtype=jnp.float32)] * NCAND
    for c in range(DEC_N // 128):
        x = delta[:, 128 * c:128 * (c + 1)]
        for j in range(NCAND):
            lo = jnp.minimum(lanes[j], x)
            x = jnp.maximum(lanes[j], x)
            lanes[j] = lo
    cand = jnp.concatenate(lanes, axis=1)
    t = jnp.full((ROWS, 1), -jnp.inf, dtype=jnp.float32)
    for _ in range(K):
        t = jnp.min(jnp.where(cand > t, cand, jnp.inf), axis=1,
                    keepdims=True)
    cnt_strict = jnp.sum((delta < t).astype(jnp.float32), axis=1)
    bad = jnp.any(cnt_strict > 15.5)
    t_ref[...] = t

    @pl.when(bad)
    def _fallback():
        tf = jnp.full((ROWS, 1), -jnp.inf, dtype=jnp.float32)
        for _ in range(K):
            tf = jnp.min(jnp.where(delta > tf, delta, jnp.inf), axis=1,
                         keepdims=True)
        t_ref[...] = tf

    t = t_ref[...]

    s = jnp.where(delta <= t, s, -jnp.inf)
    m = jnp.max(s, axis=1, keepdims=True)
    p = jnp.exp(s - m)
    denom = jnp.sum(p, axis=1, keepdims=True)
    agg = jnp.dot(p.astype(jnp.bfloat16), V_ref[...].astype(jnp.bfloat16),
                  preferred_element_type=jnp.float32) / denom

    h = (jnp.dot(agg.astype(jnp.bfloat16), W1a_ref[...].astype(jnp.bfloat16),
                 preferred_element_type=jnp.float32)
         + jnp.dot(ef.astype(jnp.bfloat16), W1b_ref[...].astype(jnp.bfloat16),
                   preferred_element_type=jnp.float32)
         + b1_ref[...])
    h = jnp.maximum(h, 0.0)
    up = jnp.dot(h.astype(jnp.bfloat16), W2_ref[...].astype(jnp.bfloat16),
                 preferred_element_type=jnp.float32) + b2_ref[...]
    mu = jnp.mean(up, axis=1, keepdims=True)
    c = up - mu
    var = jnp.mean(c * c, axis=1, keepdims=True)
    out_ref[...] = c * jax.lax.rsqrt(var + 1e-5) * gamma_ref[...] + beta_ref[...]


def kernel(decoder_features, decoder_pos, encoder_features, encoder_pos,
           encoder_labels, Wq, bq, Wk, bk, Wv, bv, W1, b1, W2, b2, gamma, beta):
    enc_n = encoder_features.shape[0]
    out_dim = W2.shape[1]

    dn_row = jnp.sum(decoder_pos * decoder_pos, axis=1)[None, :]
    dpos_aug = jnp.concatenate(
        [decoder_pos.T.astype(jnp.bfloat16),
         jnp.zeros((1, DEC_N), jnp.bfloat16)], axis=0)
    epos_aug = jnp.concatenate(
        [(-2.0 * encoder_pos).astype(jnp.bfloat16),
         jnp.zeros((enc_n, 1), jnp.bfloat16)], axis=1)
    W1a, W1b = W1[:out_dim], W1[out_dim:]
    row = lambda v: v.reshape(1, -1)

    K_dec, V_dec = pl.pallas_call(
        _precompute_kv,
        out_shape=[jax.ShapeDtypeStruct((DEC_N, out_dim), jnp.float32)] * 2,
    )(decoder_features, Wk, row(bk), Wv, row(bv))

    grid = enc_n // ROWS
    out = pl.pallas_call(
        _main,
        grid=(grid,),
        in_specs=[
            pl.BlockSpec((ROWS, 4), lambda i: (i, 0)),
            pl.BlockSpec((ROWS, 128), lambda i: (i, 0)),
            pl.BlockSpec((4, DEC_N), lambda i: (0, 0)),
            pl.BlockSpec((1, DEC_N), lambda i: (0, 0)),
            pl.BlockSpec((DEC_N, out_dim), lambda i: (0, 0)),
            pl.BlockSpec((DEC_N, out_dim), lambda i: (0, 0)),
            pl.BlockSpec((128, 256), lambda i: (0, 0)),
            pl.BlockSpec((1, 256), lambda i: (0, 0)),
            pl.BlockSpec((256, 256), lambda i: (0, 0)),
            pl.BlockSpec((128, 256), lambda i: (0, 0)),
            pl.BlockSpec((1, 256), lambda i: (0, 0)),
            pl.BlockSpec((256, 256), lambda i: (0, 0)),
            pl.BlockSpec((1, 256), lambda i: (0, 0)),
            pl.BlockSpec((1, 256), lambda i: (0, 0)),
            pl.BlockSpec((1, 256), lambda i: (0, 0)),
        ],
        out_specs=pl.BlockSpec((ROWS, out_dim), lambda i: (i, 0)),
        out_shape=jax.ShapeDtypeStruct((enc_n, out_dim), jnp.float32),
        scratch_shapes=[pltpu.VMEM((ROWS, 1), jnp.float32)],
    )(epos_aug, encoder_features, dpos_aug, dn_row, K_dec, V_dec, Wq, row(bq),
      W1a, W1b, row(b1), W2, row(b2), row(gamma), row(beta))

    return (out, encoder_pos, encoder_labels)


# speedup vs baseline: 26.5464x; 1.0869x over previous
import math
import jax
import jax.numpy as jnp
from jax.experimental import pallas as pl
from jax.experimental.pallas import tpu as pltpu

ROWS = 512
DEC_N = 4096
K = 16
SCALE = 1.0 / 16.0


def _precompute_kv(df_ref, Wk_ref, bk_ref, Wv_ref, bv_ref, K_ref, V_ref):
    df = df_ref[...]
    K_ref[...] = jnp.dot(df, Wk_ref[...], preferred_element_type=jnp.float32) + bk_ref[...]
    V_ref[...] = jnp.dot(df, Wv_ref[...], preferred_element_type=jnp.float32) + bv_ref[...]


def _main(epos_ref, efeat_ref, dpos_ref, dn_ref, K_ref, V_ref, Wq_ref, bq_ref,
          W1a_ref, W1b_ref, b1_ref, W2_ref, b2_ref, gamma_ref, beta_ref,
          out_ref, t_ref):
    ef = efeat_ref[...]
    delta = jax.lax.dot_general(
        epos_ref[...], dpos_ref[...], (((1,), (0,)), ((), ())),
        preferred_element_type=jnp.float32) + dn_ref[...]

    q = jnp.dot(ef, Wq_ref[...], preferred_element_type=jnp.float32) + bq_ref[...]
    s = jax.lax.dot_general(
        q.astype(jnp.bfloat16), K_ref[...].astype(jnp.bfloat16),
        (((1,), (1,)), ((), ())),
        preferred_element_type=jnp.float32) * SCALE

    NCAND = 5
    lanes = [jnp.full((ROWS, 128), jnp.inf, dtype=jnp.float32)] * NCAND
    for c in range(DEC_N // 128):
        x = delta[:, 128 * c:128 * (c + 1)]
        for j in range(NCAND):
            lo = jnp.minimum(lanes[j], x)
            x = jnp.maximum(lanes[j], x)
            lanes[j] = lo
    cand = jnp.concatenate(lanes, axis=1)
    t = jnp.full((ROWS, 1), -jnp.inf, dtype=jnp.float32)
    for _ in range(K):
        t = jnp.min(jnp.where(cand > t, cand, jnp.inf), axis=1,
                    keepdims=True)
    cnt_strict = jnp.sum((delta < t).astype(jnp.float32), axis=1)
    bad = jnp.any(cnt_strict > 15.5)
    t_ref[...] = t

    @pl.when(bad)
    def _fallback():
        tf = jnp.full((ROWS, 1), -jnp.inf, dtype=jnp.float32)
        for _ in range(K):
            tf = jnp.min(jnp.where(delta > tf, delta, jnp.inf), axis=1,
                         keepdims=True)
        t_ref[...] = tf

    t = t_ref[...]

    s = jnp.where(delta <= t, s, -jnp.inf)
    m = jnp.max(s, axis=1, keepdims=True)
    p = jnp.exp(s - m)
    denom = jnp.sum(p, axis=1, keepdims=True)
    agg = jnp.dot(p.astype(jnp.bfloat16), V_ref[...].astype(jnp.bfloat16),
                  preferred_element_type=jnp.float32) / denom

    h = (jnp.dot(agg.astype(jnp.bfloat16), W1a_ref[...].astype(jnp.bfloat16),
                 preferred_element_type=jnp.float32)
         + jnp.dot(ef.astype(jnp.bfloat16), W1b_ref[...].astype(jnp.bfloat16),
                   preferred_element_type=jnp.float32)
         + b1_ref[...])
    h = jnp.maximum(h, 0.0)
    up = jnp.dot(h.astype(jnp.bfloat16), W2_ref[...].astype(jnp.bfloat16),
                 preferred_element_type=jnp.float32) + b2_ref[...]
    mu = jnp.mean(up, axis=1, keepdims=True)
    c = up - mu
    var = jnp.mean(c * c, axis=1, keepdims=True)
    out_ref[...] = c * jax.lax.rsqrt(var + 1e-5) * gamma_ref[...] + beta_ref[...]


def kernel(decoder_features, decoder_pos, encoder_features, encoder_pos,
           encoder_labels, Wq, bq, Wk, bk, Wv, bv, W1, b1, W2, b2, gamma, beta):
    enc_n = encoder_features.shape[0]
    out_dim = W2.shape[1]

    dn_row = jnp.sum(decoder_pos * decoder_pos, axis=1)[None, :]
    dpos_aug = jnp.concatenate(
        [decoder_pos.T.astype(jnp.bfloat16),
         jnp.zeros((1, DEC_N), jnp.bfloat16)], axis=0)
    epos_aug = jnp.concatenate(
        [(-2.0 * encoder_pos).astype(jnp.bfloat16),
         jnp.zeros((enc_n, 1), jnp.bfloat16)], axis=1)
    W1a, W1b = W1[:out_dim], W1[out_dim:]
    row = lambda v: v.reshape(1, -1)

    K_dec, V_dec = pl.pallas_call(
        _precompute_kv,
        out_shape=[jax.ShapeDtypeStruct((DEC_N, out_dim), jnp.float32)] * 2,
    )(decoder_features, Wk, row(bk), Wv, row(bv))

    grid = enc_n // ROWS
    out = pl.pallas_call(
        _main,
        grid=(grid,),
        in_specs=[
            pl.BlockSpec((ROWS, 4), lambda i: (i, 0)),
            pl.BlockSpec((ROWS, 128), lambda i: (i, 0)),
            pl.BlockSpec((4, DEC_N), lambda i: (0, 0)),
            pl.BlockSpec((1, DEC_N), lambda i: (0, 0)),
            pl.BlockSpec((DEC_N, out_dim), lambda i: (0, 0)),
            pl.BlockSpec((DEC_N, out_dim), lambda i: (0, 0)),
            pl.BlockSpec((128, 256), lambda i: (0, 0)),
            pl.BlockSpec((1, 256), lambda i: (0, 0)),
            pl.BlockSpec((256, 256), lambda i: (0, 0)),
            pl.BlockSpec((128, 256), lambda i: (0, 0)),
            pl.BlockSpec((1, 256), lambda i: (0, 0)),
            pl.BlockSpec((256, 256), lambda i: (0, 0)),
            pl.BlockSpec((1, 256), lambda i: (0, 0)),
            pl.BlockSpec((1, 256), lambda i: (0, 0)),
            pl.BlockSpec((1, 256), lambda i: (0, 0)),
        ],
        out_specs=pl.BlockSpec((ROWS, out_dim), lambda i: (i, 0)),
        out_shape=jax.ShapeDtypeStruct((enc_n, out_dim), jnp.float32),
        scratch_shapes=[pltpu.VMEM((ROWS, 1), jnp.float32)],
    )(epos_aug, encoder_features, dpos_aug, dn_row, K_dec, V_dec, Wq, row(bq),
      W1a, W1b, row(b1), W2, row(b2), row(gamma), row(beta))

    return (out, encoder_pos, encoder_labels)
